# baseline (speedup 1.0000x reference)
import functools

import jax
import jax.numpy as jnp
from jax import lax
from jax.experimental import pallas as pl
from jax.experimental.pallas import tpu as pltpu
from jax.experimental.pallas import tpu_sc as plsc

N = 10000
E = 320000
D = 128

NC = 2
NT = 16
NP = 10240
HALF = NP // NC
SL = HALF // NT
EPT = E // NT
GCH = 64


_BLKN = 1024


def _norm_body(f_ref, o_ref):
    x = f_ref[...]
    ss = jnp.sum(x * x, axis=1)
    o_ref[...] = jnp.sqrt(ss).reshape(_BLKN // 128, 128)


def _norms(feature_pad):
    out = pl.pallas_call(
        _norm_body,
        grid=(NP // _BLKN,),
        in_specs=[pl.BlockSpec((_BLKN, D), lambda i: (i, 0))],
        out_specs=pl.BlockSpec((_BLKN // 128, 128), lambda i: (i, 0)),
        out_shape=jax.ShapeDtypeStruct((NP // 128, 128), jnp.float32),
    )(feature_pad)
    return out.reshape(NP)


def _agg_body(src_hbm, dst_hbm, norms_hbm, feat_hbm,
              h_agg, pmax, psel, psrc, mmax,
              src_c, dst_c, norms_v, maxloc, maxlocB, selloc, sellocB,
              srcloc, srclocB, mbB, miB, msB, idxb, rows, sem):
    c = lax.axis_index("c")
    s = lax.axis_index("s")
    wid = c * NT + s
    base = c * HALF + s * SL
    iota16 = lax.iota(jnp.int32, 16)

    pltpu.sync_copy(src_hbm.at[pl.ds(s * EPT, EPT)], src_c)
    pltpu.sync_copy(dst_hbm.at[pl.ds(s * EPT, EPT)], dst_c)
    pltpu.sync_copy(norms_hbm, norms_v)

    def init_body(i, _):
        sl = pl.ds(i * 16, 16)
        maxloc[sl] = jnp.full((16,), -1.0, jnp.float32)
        maxlocB[sl] = jnp.full((16,), -1.0, jnp.float32)
        selloc[sl] = jnp.full((16,), E, jnp.int32)
        sellocB[sl] = jnp.full((16,), E, jnp.int32)
        return 0

    lax.fori_loop(0, NP // 16, init_body, 0)

    def p1_one(ch, acc):
        sl = pl.ds(ch * 16, 16)
        s16 = src_c[sl]
        d16 = dst_c[sl]
        en = plsc.load_gather(norms_v, [s16])
        en_s, d_s = plsc.sort_key_val(en, d16)
        _, last = plsc.scan_count(d_s)
        cur = plsc.load_gather(acc, [d_s])
        plsc.store_scatter(acc, [d_s], jnp.maximum(cur, en_s), mask=last)

    def p1(i, _):
        p1_one(2 * i, maxloc)
        p1_one(2 * i + 1, maxlocB)
        return 0

    lax.fori_loop(0, EPT // 32, p1, 0)

    def p1_fold(v, _):
        sl = pl.ds(v * 16, 16)
        maxloc[sl] = jnp.maximum(maxloc[sl], maxlocB[sl])
        return 0

    lax.fori_loop(0, NP // 16, p1_fold, 0)

    pltpu.sync_copy(maxloc, pmax.at[pl.ds(wid * NP, NP)])
    plsc.subcore_barrier()

    def acc_max_v(v, _):
        sl = pl.ds(v * 16, 16)
        off = pl.ds(base + v * 16, 16)
        maxloc[off] = jnp.maximum(maxloc[off], mbB[sl])
        return 0

    def mk(k, _):
        pltpu.sync_copy(pmax.at[pl.ds((c * NT + k) * NP + base, SL)], mbB)
        lax.fori_loop(0, SL // 16, acc_max_v, 0)
        return 0

    lax.fori_loop(0, NT, mk, 0)

    def wr_merged(v, _):
        sl = pl.ds(v * 16, 16)
        off = pl.ds(base + v * 16, 16)
        mbB[sl] = maxloc[off]
        return 0

    lax.fori_loop(0, SL // 16, wr_merged, 0)
    pltpu.sync_copy(mbB, mmax.at[pl.ds(base, SL)])
    plsc.subcore_barrier()
    pltpu.sync_copy(mmax.at[pl.ds(c * HALF, HALF)],
                    maxloc.at[pl.ds(c * HALF, HALF)])

    def p2_one(ch, sacc, cacc):
        sl = pl.ds(ch * 16, 16)
        s16 = src_c[sl]
        d16 = dst_c[sl]
        en = plsc.load_gather(norms_v, [s16])
        gm = plsc.load_gather(maxloc, [d16])
        eid = s * EPT + ch * 16 + iota16
        cand = jnp.where(en == gm, eid, jnp.int32(E))
        cand_s, d_s = plsc.sort_key_val(cand, d16, descending=True)
        _, src_s = plsc.sort_key_val(cand, s16, descending=True)
        _, last = plsc.scan_count(d_s)
        cur = plsc.load_gather(sacc, [d_s])
        m = last & (cand_s < cur)
        plsc.store_scatter(sacc, [d_s], cand_s, mask=m)
        plsc.store_scatter(cacc, [d_s], src_s, mask=m)

    def p2(i, _):
        p2_one(2 * i, selloc, srcloc)
        p2_one(2 * i + 1, sellocB, srclocB)
        return 0

    lax.fori_loop(0, EPT // 32, p2, 0)

    def p2_fold(v, _):
        sl = pl.ds(v * 16, 16)
        pv = sellocB[sl]
        av = selloc[sl]
        better = pv < av
        selloc[sl] = jnp.where(better, pv, av)
        srcloc[sl] = jnp.where(better, srclocB[sl], srcloc[sl])
        return 0

    lax.fori_loop(0, NP // 16, p2_fold, 0)

    pltpu.sync_copy(selloc, psel.at[pl.ds(wid * NP, NP)])
    pltpu.sync_copy(srcloc, psrc.at[pl.ds(wid * NP, NP)])
    plsc.subcore_barrier()

    def acc_sel_v(v, _):
        sl = pl.ds(v * 16, 16)
        off = pl.ds(base + v * 16, 16)
        pv = miB[sl]
        av = selloc[off]
        better = pv < av
        selloc[off] = jnp.where(better, pv, av)
        srcloc[off] = jnp.where(better, msB[sl], srcloc[off])
        return 0

    def mk2(k, _):
        pltpu.sync_copy(psel.at[pl.ds((c * NT + k) * NP + base, SL)], miB)
        pltpu.sync_copy(psrc.at[pl.ds((c * NT + k) * NP + base, SL)], msB)
        lax.fori_loop(0, SL // 16, acc_sel_v, 0)
        return 0

    lax.fori_loop(0, NT, mk2, 0)

    def p3a(v, _):
        sl = pl.ds(v * 16, 16)
        off = pl.ds(base + v * 16, 16)
        has = selloc[off] < E
        nid = base + v * 16 + iota16
        idxb[sl] = jnp.where(has, srcloc[off], nid)
        return 0

    lax.fori_loop(0, SL // 16, p3a, 0)

    for g in range(SL // GCH):
        cp = pltpu.async_copy(feat_hbm.at[idxb.at[pl.ds(g * GCH, GCH)]],
                              rows, sem)
        cp.wait()
        pltpu.sync_copy(rows, h_agg.at[pl.ds(base + g * GCH, GCH)])


@functools.lru_cache(maxsize=1)
def _make_agg():
    mesh = plsc.VectorSubcoreMesh(core_axis_name="c", subcore_axis_name="s",
                                  num_cores=NC, num_subcores=NT)
    return pl.kernel(
        _agg_body,
        out_type=[
            jax.ShapeDtypeStruct((NP, D), jnp.float32),
            jax.ShapeDtypeStruct((NC * NT * NP,), jnp.float32),
            jax.ShapeDtypeStruct((NC * NT * NP,), jnp.int32),
            jax.ShapeDtypeStruct((NC * NT * NP,), jnp.int32),
            jax.ShapeDtypeStruct((NP,), jnp.float32),
        ],
        mesh=mesh,
        compiler_params=pltpu.CompilerParams(needs_layout_passes=False),
        scratch_types=[
            pltpu.VMEM((EPT,), jnp.int32),
            pltpu.VMEM((EPT,), jnp.int32),
            pltpu.VMEM((NP,), jnp.float32),
            pltpu.VMEM((NP,), jnp.float32),
            pltpu.VMEM((NP,), jnp.float32),
            pltpu.VMEM((NP,), jnp.int32),
            pltpu.VMEM((NP,), jnp.int32),
            pltpu.VMEM((NP,), jnp.int32),
            pltpu.VMEM((NP,), jnp.int32),
            pltpu.VMEM((SL,), jnp.float32),
            pltpu.VMEM((SL,), jnp.int32),
            pltpu.VMEM((SL,), jnp.int32),
            pltpu.VMEM((SL,), jnp.int32),
            pltpu.VMEM((GCH, D), jnp.float32),
            pltpu.SemaphoreType.DMA,
        ],
    )


_BLKL = 2000


def _cell(x, hp, cp, wT, whT, bih, bhh):
    g = (jnp.dot(x, wT, preferred_element_type=jnp.float32)
         + jnp.dot(hp, whT, preferred_element_type=jnp.float32)
         + bih + bhh)
    i = jax.nn.sigmoid(g[:, 0:D])
    f = jax.nn.sigmoid(g[:, D:2 * D])
    gg = jnp.tanh(g[:, 2 * D:3 * D])
    o = jax.nn.sigmoid(g[:, 3 * D:4 * D])
    cc = f * cp + i * gg
    h = o * jnp.tanh(cc)
    return h, cc


def _lstm_body(ha, ft, h0r, c0r,
               w0, wh0, bi0, bh0, w1, wh1, bi1, bh1, w2, wh2, bi2, bh2,
               out_r, hn_r, cn_r):
    x = jnp.concatenate([ha[...], ft[...]], axis=1)
    h1, c1 = _cell(x, h0r[0], c0r[0], w0[...], wh0[...], bi0[...], bh0[...])
    h2, c2 = _cell(h1, h0r[1], c0r[1], w1[...], wh1[...], bi1[...], bh1[...])
    h3, c3 = _cell(h2, h0r[2], c0r[2], w2[...], wh2[...], bi2[...], bh2[...])
    hn_r[0] = h1
    hn_r[1] = h2
    hn_r[2] = h3
    cn_r[0] = c1
    cn_r[1] = c2
    cn_r[2] = c3
    out_r[0] = h3


def _lstm(h_agg, feature, h0, c0, ws):
    row_spec = pl.BlockSpec((_BLKL, D), lambda i: (i, 0))
    stk_spec = pl.BlockSpec((3, _BLKL, D), lambda i: (0, i, 0))

    def w_spec(shape):
        return pl.BlockSpec(shape, lambda i: tuple(0 for _ in shape))

    in_specs = [row_spec, row_spec, stk_spec, stk_spec]
    w_args = []
    for (wih, whh, bih, bhh) in ws:
        wihT = wih.T
        whhT = whh.T
        bi = bih.reshape(1, 4 * D)
        bh = bhh.reshape(1, 4 * D)
        for a in (wihT, whhT, bi, bh):
            in_specs.append(w_spec(a.shape))
            w_args.append(a)

    out, h_n, c_n = pl.pallas_call(
        _lstm_body,
        grid=(N // _BLKL,),
        in_specs=in_specs,
        out_specs=[
            pl.BlockSpec((1, _BLKL, D), lambda i: (0, i, 0)),
            stk_spec,
            stk_spec,
        ],
        out_shape=[
            jax.ShapeDtypeStruct((1, N, D), jnp.float32),
            jax.ShapeDtypeStruct((3, N, D), jnp.float32),
            jax.ShapeDtypeStruct((3, N, D), jnp.float32),
        ],
    )(h_agg, feature, h0, c0, *w_args)
    return out, h_n, c_n


def kernel(feature, edge_index, h0, c0,
           W_ih_0, W_hh_0, b_ih_0, b_hh_0,
           W_ih_1, W_hh_1, b_ih_1, b_hh_1,
           W_ih_2, W_hh_2, b_ih_2, b_hh_2):
    src = edge_index[0]
    dst = edge_index[1]
    fpad = jnp.pad(feature, ((0, NP - N), (0, 0)))
    norms = _norms(fpad)
    h_agg = _make_agg()(src, dst, norms, fpad)[0]
    ws = [(W_ih_0, W_hh_0, b_ih_0, b_hh_0),
          (W_ih_1, W_hh_1, b_ih_1, b_hh_1),
          (W_ih_2, W_hh_2, b_ih_2, b_hh_2)]
    return _lstm(h_agg, feature, h0, c0, ws)


# speedup vs baseline: 1.1468x; 1.1468x over previous
import functools

import jax
import jax.numpy as jnp
from jax import lax
from jax.experimental import pallas as pl
from jax.experimental.pallas import tpu as pltpu
from jax.experimental.pallas import tpu_sc as plsc

N = 10000
E = 320000
D = 128

NC = 2
NT = 16
NP = 10240
HALF = NP // NC
SL = HALF // NT
EPT = E // NT
GCH = 64


_BLKN = 1024


def _norm_body(f_ref, o_ref):
    x = f_ref[...]
    ss = jnp.sum(x * x, axis=1)
    o_ref[...] = jnp.sqrt(ss).reshape(_BLKN // 128, 128)


def _norms(feature_pad):
    out = pl.pallas_call(
        _norm_body,
        grid=(NP // _BLKN,),
        in_specs=[pl.BlockSpec((_BLKN, D), lambda i: (i, 0))],
        out_specs=pl.BlockSpec((_BLKN // 128, 128), lambda i: (i, 0)),
        out_shape=jax.ShapeDtypeStruct((NP // 128, 128), jnp.float32),
    )(feature_pad)
    return out.reshape(NP)


def _agg_body(src_hbm, dst_hbm, norms_hbm, feat_hbm,
              h_agg, pmax, psel, psrc, mmax,
              src_c, dst_c, norms_v, maxloc, selloc, srcloc,
              mstage, istage, sstage, idxb, rowsA, rowsB, sem):
    c = lax.axis_index("c")
    s = lax.axis_index("s")
    wid = c * NT + s
    base = c * HALF + s * SL
    iota16 = lax.iota(jnp.int32, 16)

    cp0 = pltpu.async_copy(src_hbm.at[pl.ds(s * EPT, EPT)], src_c, sem)
    cp1 = pltpu.async_copy(dst_hbm.at[pl.ds(s * EPT, EPT)], dst_c, sem)
    cp2 = pltpu.async_copy(norms_hbm, norms_v, sem)
    cp0.wait()
    cp1.wait()
    cp2.wait()

    def init_body(i, _):
        sl = pl.ds(i * 16, 16)
        maxloc[sl] = jnp.full((16,), -1.0, jnp.float32)
        selloc[sl] = jnp.full((16,), E, jnp.int32)
        return 0

    lax.fori_loop(0, NP // 16, init_body, 0)

    def p1(i, _):
        sl = pl.ds(i * 16, 16)
        s16 = src_c[sl]
        d16 = dst_c[sl]
        en = plsc.load_gather(norms_v, [s16])
        en_s, d_s = plsc.sort_key_val(en, d16)
        _, last = plsc.scan_count(d_s)
        cur = plsc.load_gather(maxloc, [d_s])
        plsc.store_scatter(maxloc, [d_s], jnp.maximum(cur, en_s), mask=last)
        return 0

    lax.fori_loop(0, EPT // 16, p1, 0)

    pltpu.sync_copy(maxloc, pmax.at[pl.ds(wid * NP, NP)])
    plsc.subcore_barrier()

    cps = []
    for k in range(NT):
        cps.append(pltpu.async_copy(
            pmax.at[pl.ds((c * NT + k) * NP + base, SL)],
            mstage.at[pl.ds(k * SL, SL)], sem))
    for cp in cps:
        cp.wait()

    def acc_max_v(v, _):
        off = pl.ds(base + (v % (SL // 16)) * 16, 16)
        sl = pl.ds((v // (SL // 16)) * SL + (v % (SL // 16)) * 16, 16)
        maxloc[off] = jnp.maximum(maxloc[off], mstage[sl])
        return 0

    lax.fori_loop(0, NT * (SL // 16), acc_max_v, 0)

    def wr_merged(v, _):
        sl = pl.ds(v * 16, 16)
        off = pl.ds(base + v * 16, 16)
        mstage[sl] = maxloc[off]
        return 0

    lax.fori_loop(0, SL // 16, wr_merged, 0)
    pltpu.sync_copy(mstage.at[pl.ds(0, SL)], mmax.at[pl.ds(base, SL)])
    plsc.subcore_barrier()
    pltpu.sync_copy(mmax.at[pl.ds(c * HALF, HALF)],
                    maxloc.at[pl.ds(c * HALF, HALF)])

    def p2(i, _):
        sl = pl.ds(i * 16, 16)
        s16 = src_c[sl]
        d16 = dst_c[sl]
        en = plsc.load_gather(norms_v, [s16])
        gm = plsc.load_gather(maxloc, [d16])
        eid = s * EPT + i * 16 + iota16
        cand = jnp.where(en == gm, eid, jnp.int32(E))
        pack = d16 | (s16 << 14)
        cand_s, pk = plsc.sort_key_val(cand, pack, descending=True)
        d_s = pk & 16383
        src_s = pk >> 14
        _, last = plsc.scan_count(d_s)
        cur = plsc.load_gather(selloc, [d_s])
        mm = last & (cand_s < cur)
        plsc.store_scatter(selloc, [d_s], cand_s, mask=mm)
        plsc.store_scatter(srcloc, [d_s], src_s, mask=mm)
        return 0

    lax.fori_loop(0, EPT // 16, p2, 0)

    pltpu.sync_copy(selloc, psel.at[pl.ds(wid * NP, NP)])
    pltpu.sync_copy(srcloc, psrc.at[pl.ds(wid * NP, NP)])
    plsc.subcore_barrier()

    cps = []
    for k in range(NT):
        cps.append(pltpu.async_copy(
            psel.at[pl.ds((c * NT + k) * NP + base, SL)],
            istage.at[pl.ds(k * SL, SL)], sem))
        cps.append(pltpu.async_copy(
            psrc.at[pl.ds((c * NT + k) * NP + base, SL)],
            sstage.at[pl.ds(k * SL, SL)], sem))
    for cp in cps:
        cp.wait()

    def acc_sel_v(v, _):
        off = pl.ds(base + (v % (SL // 16)) * 16, 16)
        sl = pl.ds((v // (SL // 16)) * SL + (v % (SL // 16)) * 16, 16)
        pv = istage[sl]
        av = selloc[off]
        better = pv < av
        selloc[off] = jnp.where(better, pv, av)
        srcloc[off] = jnp.where(better, sstage[sl], srcloc[off])
        return 0

    lax.fori_loop(0, NT * (SL // 16), acc_sel_v, 0)

    def p3a(v, _):
        sl = pl.ds(v * 16, 16)
        off = pl.ds(base + v * 16, 16)
        has = selloc[off] < E
        nid = base + v * 16 + iota16
        idxb[sl] = jnp.where(has, srcloc[off], nid)
        return 0

    lax.fori_loop(0, SL // 16, p3a, 0)

    bufs = [rowsA, rowsB]
    ng = SL // GCH
    cps = [pltpu.async_copy(feat_hbm.at[idxb.at[pl.ds(0, GCH)]], rowsA, sem)]
    for g in range(ng):
        cps[g].wait()
        if g + 1 < ng:
            cps.append(pltpu.async_copy(
                feat_hbm.at[idxb.at[pl.ds((g + 1) * GCH, GCH)]],
                bufs[(g + 1) % 2], sem))
        pltpu.sync_copy(bufs[g % 2], h_agg.at[pl.ds(base + g * GCH, GCH)])


@functools.lru_cache(maxsize=1)
def _make_agg():
    mesh = plsc.VectorSubcoreMesh(core_axis_name="c", subcore_axis_name="s",
                                  num_cores=NC, num_subcores=NT)
    return pl.kernel(
        _agg_body,
        out_type=[
            jax.ShapeDtypeStruct((NP, D), jnp.float32),
            jax.ShapeDtypeStruct((NC * NT * NP,), jnp.float32),
            jax.ShapeDtypeStruct((NC * NT * NP,), jnp.int32),
            jax.ShapeDtypeStruct((NC * NT * NP,), jnp.int32),
            jax.ShapeDtypeStruct((NP,), jnp.float32),
        ],
        mesh=mesh,
        compiler_params=pltpu.CompilerParams(needs_layout_passes=False),
        scratch_types=[
            pltpu.VMEM((EPT,), jnp.int32),
            pltpu.VMEM((EPT,), jnp.int32),
            pltpu.VMEM((NP,), jnp.float32),
            pltpu.VMEM((NP,), jnp.float32),
            pltpu.VMEM((NP,), jnp.int32),
            pltpu.VMEM((NP,), jnp.int32),
            pltpu.VMEM((NT * SL,), jnp.float32),
            pltpu.VMEM((NT * SL,), jnp.int32),
            pltpu.VMEM((NT * SL,), jnp.int32),
            pltpu.VMEM((SL,), jnp.int32),
            pltpu.VMEM((GCH, D), jnp.float32),
            pltpu.VMEM((GCH, D), jnp.float32),
            pltpu.SemaphoreType.DMA,
        ],
    )


_BLKL = 2000


def _cell(x, hp, cp, wT, whT, bih, bhh):
    g = (jnp.dot(x, wT, preferred_element_type=jnp.float32)
         + jnp.dot(hp, whT, preferred_element_type=jnp.float32)
         + bih + bhh)
    i = jax.nn.sigmoid(g[:, 0:D])
    f = jax.nn.sigmoid(g[:, D:2 * D])
    gg = jnp.tanh(g[:, 2 * D:3 * D])
    o = jax.nn.sigmoid(g[:, 3 * D:4 * D])
    cc = f * cp + i * gg
    h = o * jnp.tanh(cc)
    return h, cc


def _lstm_body(ha, ft, h0r, c0r,
               w0, wh0, bi0, bh0, w1, wh1, bi1, bh1, w2, wh2, bi2, bh2,
               out_r, hn_r, cn_r):
    x = jnp.concatenate([ha[...], ft[...]], axis=1)
    h1, c1 = _cell(x, h0r[0], c0r[0], w0[...], wh0[...], bi0[...], bh0[...])
    h2, c2 = _cell(h1, h0r[1], c0r[1], w1[...], wh1[...], bi1[...], bh1[...])
    h3, c3 = _cell(h2, h0r[2], c0r[2], w2[...], wh2[...], bi2[...], bh2[...])
    hn_r[0] = h1
    hn_r[1] = h2
    hn_r[2] = h3
    cn_r[0] = c1
    cn_r[1] = c2
    cn_r[2] = c3
    out_r[0] = h3


def _lstm(h_agg, feature, h0, c0, ws):
    row_spec = pl.BlockSpec((_BLKL, D), lambda i: (i, 0))
    stk_spec = pl.BlockSpec((3, _BLKL, D), lambda i: (0, i, 0))

    def w_spec(shape):
        return pl.BlockSpec(shape, lambda i: tuple(0 for _ in shape))

    in_specs = [row_spec, row_spec, stk_spec, stk_spec]
    w_args = []
    for (wih, whh, bih, bhh) in ws:
        wihT = wih.T
        whhT = whh.T
        bi = bih.reshape(1, 4 * D)
        bh = bhh.reshape(1, 4 * D)
        for a in (wihT, whhT, bi, bh):
            in_specs.append(w_spec(a.shape))
            w_args.append(a)

    out, h_n, c_n = pl.pallas_call(
        _lstm_body,
        grid=(N // _BLKL,),
        in_specs=in_specs,
        out_specs=[
            pl.BlockSpec((1, _BLKL, D), lambda i: (0, i, 0)),
            stk_spec,
            stk_spec,
        ],
        out_shape=[
            jax.ShapeDtypeStruct((1, N, D), jnp.float32),
            jax.ShapeDtypeStruct((3, N, D), jnp.float32),
            jax.ShapeDtypeStruct((3, N, D), jnp.float32),
        ],
    )(h_agg, feature, h0, c0, *w_args)
    return out, h_n, c_n


def kernel(feature, edge_index, h0, c0,
           W_ih_0, W_hh_0, b_ih_0, b_hh_0,
           W_ih_1, W_hh_1, b_ih_1, b_hh_1,
           W_ih_2, W_hh_2, b_ih_2, b_hh_2):
    src = edge_index[0]
    dst = edge_index[1]
    fpad = jnp.pad(feature, ((0, NP - N), (0, 0)))
    norms = _norms(fpad)
    h_agg = _make_agg()(src, dst, norms, fpad)[0]
    ws = [(W_ih_0, W_hh_0, b_ih_0, b_hh_0),
          (W_ih_1, W_hh_1, b_ih_1, b_hh_1),
          (W_ih_2, W_hh_2, b_ih_2, b_hh_2)]
    return _lstm(h_agg, feature, h0, c0, ws)


# speedup vs baseline: 1.4324x; 1.2490x over previous
import functools

import jax
import jax.numpy as jnp
from jax import lax
from jax.experimental import pallas as pl
from jax.experimental.pallas import tpu as pltpu
from jax.experimental.pallas import tpu_sc as plsc

N = 10000
E = 320000
D = 128

NC = 2
NT = 16
NP = 10240
HALF = NP // NC
SL = HALF // NT
EPT = E // NT
GCH = 64


_BLKN = 1024


def _norm_body(f_ref, o_ref):
    x = f_ref[...]
    ss = jnp.sum(x * x, axis=1)
    o_ref[...] = jnp.sqrt(ss).reshape(_BLKN // 128, 128)


def _norms(feature_pad):
    out = pl.pallas_call(
        _norm_body,
        grid=(NP // _BLKN,),
        in_specs=[pl.BlockSpec((_BLKN, D), lambda i: (i, 0))],
        out_specs=pl.BlockSpec((_BLKN // 128, 128), lambda i: (i, 0)),
        out_shape=jax.ShapeDtypeStruct((NP // 128, 128), jnp.float32),
    )(feature_pad)
    return out.reshape(NP)


SEGC = 50
NSEG = EPT // 16 // SEGC
DEAD = NP


def _agg_body(src_hbm, dst_hbm, norms_hbm, feat_hbm,
              h_agg, pmax, psel, psrc, mmax,
              src_c, dst_c, norms_v, maxloc, selloc, srcloc,
              mstage, istage, sstage, idxb, rowsA, rowsB,
              bufi, buff, bufp, sem):
    c = lax.axis_index("c")
    s = lax.axis_index("s")
    wid = c * NT + s
    base = c * HALF + s * SL
    iota16 = lax.iota(jnp.int32, 16)

    cp0 = pltpu.async_copy(src_hbm.at[pl.ds(s * EPT, EPT)], src_c, sem)
    cp1 = pltpu.async_copy(dst_hbm.at[pl.ds(s * EPT, EPT)], dst_c, sem)
    cp2 = pltpu.async_copy(norms_hbm, norms_v, sem)
    cp0.wait()
    cp1.wait()
    cp2.wait()

    def init_body(i, _):
        sl = pl.ds(i * 16, 16)
        maxloc[sl] = jnp.full((16,), -1.0, jnp.float32)
        selloc[sl] = jnp.full((16,), E, jnp.int32)
        return 0

    lax.fori_loop(0, NP // 16 + 1, init_body, 0)

    def p1a_one(ch, bo):
        sl = pl.ds(ch * 16, 16)
        s16 = src_c[sl]
        d16 = dst_c[sl]
        en = plsc.load_gather(norms_v, [s16])
        en_s, d_s = plsc.sort_key_val(en, d16)
        _, last = plsc.scan_count(d_s)
        bsl = pl.ds(bo, 16)
        bufi[bsl] = jnp.where(last, d_s, jnp.int32(DEAD))
        buff[bsl] = en_s

    def p1b_one(bo):
        bsl = pl.ds(bo, 16)
        dv = bufi[bsl]
        ev = buff[bsl]
        cur = plsc.load_gather(maxloc, [dv])
        plsc.store_scatter(maxloc, [dv], jnp.maximum(cur, ev))

    def p1seg(q, _):
        @plsc.parallel_loop(0, SEGC, 1, unroll=4)
        def _pa(j):
            p1a_one(q * SEGC + j, 16 * j)

        def pb(j, _):
            p1b_one(16 * j)
            return 0

        lax.fori_loop(0, SEGC, pb, 0)
        return 0

    lax.fori_loop(0, NSEG, p1seg, 0)

    pltpu.sync_copy(maxloc.at[pl.ds(0, NP)], pmax.at[pl.ds(wid * NP, NP)])
    plsc.subcore_barrier()

    cps = []
    for k in range(NT):
        cps.append(pltpu.async_copy(
            pmax.at[pl.ds((c * NT + k) * NP + base, SL)],
            mstage.at[pl.ds(k * SL, SL)], sem))
    for cp in cps:
        cp.wait()

    def acc_max_v(v, _):
        off = pl.ds(base + (v % (SL // 16)) * 16, 16)
        sl = pl.ds((v // (SL // 16)) * SL + (v % (SL // 16)) * 16, 16)
        maxloc[off] = jnp.maximum(maxloc[off], mstage[sl])
        return 0

    lax.fori_loop(0, NT * (SL // 16), acc_max_v, 0)

    def wr_merged(v, _):
        sl = pl.ds(v * 16, 16)
        off = pl.ds(base + v * 16, 16)
        mstage[sl] = maxloc[off]
        return 0

    lax.fori_loop(0, SL // 16, wr_merged, 0)
    pltpu.sync_copy(mstage.at[pl.ds(0, SL)], mmax.at[pl.ds(base, SL)])
    plsc.subcore_barrier()
    pltpu.sync_copy(mmax.at[pl.ds(c * HALF, HALF)],
                    maxloc.at[pl.ds(c * HALF, HALF)])

    def p2a_one(ch, bo):
        sl = pl.ds(ch * 16, 16)
        s16 = src_c[sl]
        d16 = dst_c[sl]
        en = plsc.load_gather(norms_v, [s16])
        gm = plsc.load_gather(maxloc, [d16])
        eid = s * EPT + ch * 16 + iota16
        cand = jnp.where(en == gm, eid, jnp.int32(E))
        pack = d16 | (s16 << 14)
        cand_s, pk = plsc.sort_key_val(cand, pack, descending=True)
        _, last = plsc.scan_count(pk & 16383)
        bsl = pl.ds(bo, 16)
        bufi[bsl] = jnp.where(last, cand_s, jnp.int32(E))
        bufp[bsl] = pk

    def p2b_one(bo):
        bsl = pl.ds(bo, 16)
        cv = bufi[bsl]
        pk = bufp[bsl]
        dv = pk & 16383
        sv = pk >> 14
        cur = plsc.load_gather(selloc, [dv])
        mm = cv < cur
        plsc.store_scatter(selloc, [dv], cv, mask=mm)
        plsc.store_scatter(srcloc, [dv], sv, mask=mm)

    def p2seg(q, _):
        @plsc.parallel_loop(0, SEGC, 1, unroll=4)
        def _pa(j):
            p2a_one(q * SEGC + j, 16 * j)

        def pb(j, _):
            p2b_one(16 * j)
            return 0

        lax.fori_loop(0, SEGC, pb, 0)
        return 0

    lax.fori_loop(0, NSEG, p2seg, 0)

    pltpu.sync_copy(selloc.at[pl.ds(0, NP)], psel.at[pl.ds(wid * NP, NP)])
    pltpu.sync_copy(srcloc, psrc.at[pl.ds(wid * NP, NP)])
    plsc.subcore_barrier()

    cps = []
    for k in range(NT):
        cps.append(pltpu.async_copy(
            psel.at[pl.ds((c * NT + k) * NP + base, SL)],
            istage.at[pl.ds(k * SL, SL)], sem))
        cps.append(pltpu.async_copy(
            psrc.at[pl.ds((c * NT + k) * NP + base, SL)],
            sstage.at[pl.ds(k * SL, SL)], sem))
    for cp in cps:
        cp.wait()

    def acc_sel_v(v, _):
        off = pl.ds(base + (v % (SL // 16)) * 16, 16)
        sl = pl.ds((v // (SL // 16)) * SL + (v % (SL // 16)) * 16, 16)
        pv = istage[sl]
        av = selloc[off]
        better = pv < av
        selloc[off] = jnp.where(better, pv, av)
        srcloc[off] = jnp.where(better, sstage[sl], srcloc[off])
        return 0

    lax.fori_loop(0, NT * (SL // 16), acc_sel_v, 0)

    def p3a(v, _):
        sl = pl.ds(v * 16, 16)
        off = pl.ds(base + v * 16, 16)
        has = selloc[off] < E
        nid = base + v * 16 + iota16
        idxb[sl] = jnp.where(has, srcloc[off], nid)
        return 0

    lax.fori_loop(0, SL // 16, p3a, 0)

    bufs = [rowsA, rowsB]
    ng = SL // GCH
    cps = [pltpu.async_copy(feat_hbm.at[idxb.at[pl.ds(0, GCH)]], rowsA, sem)]
    for g in range(ng):
        cps[g].wait()
        if g + 1 < ng:
            cps.append(pltpu.async_copy(
                feat_hbm.at[idxb.at[pl.ds((g + 1) * GCH, GCH)]],
                bufs[(g + 1) % 2], sem))
        pltpu.sync_copy(bufs[g % 2], h_agg.at[pl.ds(base + g * GCH, GCH)])


@functools.lru_cache(maxsize=1)
def _make_agg():
    mesh = plsc.VectorSubcoreMesh(core_axis_name="c", subcore_axis_name="s",
                                  num_cores=NC, num_subcores=NT)
    return pl.kernel(
        _agg_body,
        out_type=[
            jax.ShapeDtypeStruct((NP, D), jnp.float32),
            jax.ShapeDtypeStruct((NC * NT * NP,), jnp.float32),
            jax.ShapeDtypeStruct((NC * NT * NP,), jnp.int32),
            jax.ShapeDtypeStruct((NC * NT * NP,), jnp.int32),
            jax.ShapeDtypeStruct((NP,), jnp.float32),
        ],
        mesh=mesh,
        compiler_params=pltpu.CompilerParams(needs_layout_passes=False),
        scratch_types=[
            pltpu.VMEM((EPT,), jnp.int32),
            pltpu.VMEM((EPT,), jnp.int32),
            pltpu.VMEM((NP,), jnp.float32),
            pltpu.VMEM((NP + 16,), jnp.float32),
            pltpu.VMEM((NP + 16,), jnp.int32),
            pltpu.VMEM((NP,), jnp.int32),
            pltpu.VMEM((NT * SL,), jnp.float32),
            pltpu.VMEM((NT * SL,), jnp.int32),
            pltpu.VMEM((NT * SL,), jnp.int32),
            pltpu.VMEM((SL,), jnp.int32),
            pltpu.VMEM((GCH, D), jnp.float32),
            pltpu.VMEM((GCH, D), jnp.float32),
            pltpu.VMEM((SEGC * 16,), jnp.int32),
            pltpu.VMEM((SEGC * 16,), jnp.float32),
            pltpu.VMEM((SEGC * 16,), jnp.int32),
            pltpu.SemaphoreType.DMA,
        ],
    )


_BLKL = 2000


def _cell(x, hp, cp, wT, whT, bih, bhh):
    g = (jnp.dot(x, wT, preferred_element_type=jnp.float32)
         + jnp.dot(hp, whT, preferred_element_type=jnp.float32)
         + bih + bhh)
    i = jax.nn.sigmoid(g[:, 0:D])
    f = jax.nn.sigmoid(g[:, D:2 * D])
    gg = jnp.tanh(g[:, 2 * D:3 * D])
    o = jax.nn.sigmoid(g[:, 3 * D:4 * D])
    cc = f * cp + i * gg
    h = o * jnp.tanh(cc)
    return h, cc


def _lstm_body(ha, ft, h0r, c0r,
               w0, wh0, bi0, bh0, w1, wh1, bi1, bh1, w2, wh2, bi2, bh2,
               out_r, hn_r, cn_r):
    x = jnp.concatenate([ha[...], ft[...]], axis=1)
    h1, c1 = _cell(x, h0r[0], c0r[0], w0[...], wh0[...], bi0[...], bh0[...])
    h2, c2 = _cell(h1, h0r[1], c0r[1], w1[...], wh1[...], bi1[...], bh1[...])
    h3, c3 = _cell(h2, h0r[2], c0r[2], w2[...], wh2[...], bi2[...], bh2[...])
    hn_r[0] = h1
    hn_r[1] = h2
    hn_r[2] = h3
    cn_r[0] = c1
    cn_r[1] = c2
    cn_r[2] = c3
    out_r[0] = h3


def _lstm(h_agg, feature, h0, c0, ws):
    row_spec = pl.BlockSpec((_BLKL, D), lambda i: (i, 0))
    stk_spec = pl.BlockSpec((3, _BLKL, D), lambda i: (0, i, 0))

    def w_spec(shape):
        return pl.BlockSpec(shape, lambda i: tuple(0 for _ in shape))

    in_specs = [row_spec, row_spec, stk_spec, stk_spec]
    w_args = []
    for (wih, whh, bih, bhh) in ws:
        wihT = wih.T
        whhT = whh.T
        bi = bih.reshape(1, 4 * D)
        bh = bhh.reshape(1, 4 * D)
        for a in (wihT, whhT, bi, bh):
            in_specs.append(w_spec(a.shape))
            w_args.append(a)

    out, h_n, c_n = pl.pallas_call(
        _lstm_body,
        grid=(N // _BLKL,),
        in_specs=in_specs,
        out_specs=[
            pl.BlockSpec((1, _BLKL, D), lambda i: (0, i, 0)),
            stk_spec,
            stk_spec,
        ],
        out_shape=[
            jax.ShapeDtypeStruct((1, N, D), jnp.float32),
            jax.ShapeDtypeStruct((3, N, D), jnp.float32),
            jax.ShapeDtypeStruct((3, N, D), jnp.float32),
        ],
    )(h_agg, feature, h0, c0, *w_args)
    return out, h_n, c_n


def kernel(feature, edge_index, h0, c0,
           W_ih_0, W_hh_0, b_ih_0, b_hh_0,
           W_ih_1, W_hh_1, b_ih_1, b_hh_1,
           W_ih_2, W_hh_2, b_ih_2, b_hh_2):
    src = edge_index[0]
    dst = edge_index[1]
    fpad = jnp.pad(feature, ((0, NP - N), (0, 0)))
    norms = _norms(fpad)
    h_agg = _make_agg()(src, dst, norms, fpad)[0]
    ws = [(W_ih_0, W_hh_0, b_ih_0, b_hh_0),
          (W_ih_1, W_hh_1, b_ih_1, b_hh_1),
          (W_ih_2, W_hh_2, b_ih_2, b_hh_2)]
    return _lstm(h_agg, feature, h0, c0, ws)
